# Initial kernel scaffold; baseline (speedup 1.0000x reference)
import jax
import jax.numpy as jnp
from jax.experimental import pallas as pl


def kernel(node_features, parent_raw, W_iou, U_iou_w, U_iou_b, W_f, U_f_w, U_f_b):
    raise NotImplementedError("write your pallas kernel here")


# speedup vs baseline: 8.6223x; 8.6223x over previous
import jax
import jax.numpy as jnp
from jax.experimental import pallas as pl

_BLK = 1024
_INTERPRET = False


def _proj_kernel(nf_ref, w_ref, b_ref, out_ref):
    out_ref[...] = (
        jnp.dot(nf_ref[...], w_ref[...], preferred_element_type=jnp.float32)
        + b_ref[...]
    )


def _level_kernel(xiou_ref, xfp_ref, hsum_ref, fc_ref, hs_ref, mf_ref, cmf_ref,
                  uiou_ref, uf_ref, hs_out_ref, hch_ref, fch_ref):
    h = hsum_ref.shape[1]
    hsum = hsum_ref[...]
    fc = fc_ref[...]
    iou = xiou_ref[...] + jnp.dot(hsum, uiou_ref[...],
                                  preferred_element_type=jnp.float32)
    i = jax.nn.sigmoid(iou[:, 0:h])
    o = jax.nn.sigmoid(iou[:, h:2 * h])
    u = jnp.tanh(iou[:, 2 * h:3 * h])
    c = i * u + fc
    hv = o * jnp.tanh(c)
    f = jax.nn.sigmoid(xfp_ref[...] + jnp.dot(hv, uf_ref[...],
                                              preferred_element_type=jnp.float32))
    m = mf_ref[...]
    cm = cmf_ref[...]
    hs_out_ref[...] = hs_ref[...] * (1.0 - m) + hv * m
    hch_ref[...] = hv * cm
    fch_ref[...] = (f * c) * cm


def kernel(node_features, parent_raw, W_iou, U_iou_w, U_iou_b, W_f, U_f_w, U_f_b):
    N, D = node_features.shape
    H = U_f_b.shape[0]
    B = _BLK
    Np = ((N + B - 1) // B) * B
    nb = Np // B
    f32 = jnp.float32

    ar = jnp.arange(N, dtype=jnp.int32)
    raw = parent_raw.astype(jnp.int32)
    parent = jnp.where(ar == 0, -1, raw % jnp.maximum(ar, 1))

    steps = max(1, int(N - 1).bit_length())

    def dbl(_, carry):
        anc, dep = carry
        a = jnp.maximum(anc, 0)
        dep = dep + jnp.where(anc >= 0, dep[a], 0)
        anc = jnp.where(anc >= 0, anc[a], -1)
        return anc, dep

    _, dep = jax.lax.fori_loop(
        0, steps, dbl, (parent, (parent >= 0).astype(jnp.int32)))
    maxd = jnp.max(dep)

    Wcat_t = jnp.concatenate([W_iou, W_f], axis=0).T
    bcat = jnp.concatenate([U_iou_b, jnp.zeros((H,), f32)]).reshape(1, 4 * H)
    nf_p = jnp.zeros((Np, D), f32).at[:N].set(node_features)

    X = pl.pallas_call(
        _proj_kernel,
        grid=(nb,),
        in_specs=[
            pl.BlockSpec((B, D), lambda i: (i, 0)),
            pl.BlockSpec((D, 4 * H), lambda i: (0, 0)),
            pl.BlockSpec((1, 4 * H), lambda i: (0, 0)),
        ],
        out_specs=pl.BlockSpec((B, 4 * H), lambda i: (i, 0)),
        out_shape=jax.ShapeDtypeStruct((Np, 4 * H), f32),
        interpret=_INTERPRET,
    )(nf_p, Wcat_t, bcat)

    Xiou = X[:, :3 * H]
    P = X[:, 3 * H:]
    pclamp = jnp.maximum(parent, 0)
    XfP = jnp.zeros((Np, H), f32).at[:N].set(P[pclamp] + U_f_b)

    dep_p = jnp.full((Np,), -1, jnp.int32).at[:N].set(dep)
    parent_p = jnp.full((Np,), -1, jnp.int32).at[:N].set(parent)
    rows = jnp.arange(Np, dtype=jnp.int32)

    UiouT = U_iou_w.T
    UfT = U_f_w.T

    level_call = pl.pallas_call(
        _level_kernel,
        grid=(nb,),
        in_specs=[
            pl.BlockSpec((B, 3 * H), lambda i: (i, 0)),
            pl.BlockSpec((B, H), lambda i: (i, 0)),
            pl.BlockSpec((B, H), lambda i: (i, 0)),
            pl.BlockSpec((B, H), lambda i: (i, 0)),
            pl.BlockSpec((B, H), lambda i: (i, 0)),
            pl.BlockSpec((B, 1), lambda i: (i, 0)),
            pl.BlockSpec((B, 1), lambda i: (i, 0)),
            pl.BlockSpec((H, 3 * H), lambda i: (0, 0)),
            pl.BlockSpec((H, H), lambda i: (0, 0)),
        ],
        out_specs=[
            pl.BlockSpec((B, H), lambda i: (i, 0)),
            pl.BlockSpec((B, H), lambda i: (i, 0)),
            pl.BlockSpec((B, H), lambda i: (i, 0)),
        ],
        out_shape=[
            jax.ShapeDtypeStruct((Np, H), f32),
            jax.ShapeDtypeStruct((Np, H), f32),
            jax.ShapeDtypeStruct((Np, H), f32),
        ],
        input_output_aliases={4: 0},
        interpret=_INTERPRET,
    )

    def body(t, carry):
        Hs, Hsum, FC = carry
        L = maxd - t
        on_level = dep_p == L
        cmb = on_level & (parent_p >= 0)
        mf = on_level.astype(f32).reshape(Np, 1)
        cmf = cmb.astype(f32).reshape(Np, 1)
        Hs, hch, fch = level_call(Xiou, XfP, Hsum, FC, Hs, mf, cmf, UiouT, UfT)
        pi = jnp.where(cmb, parent_p, rows)
        Hsum = Hsum.at[pi].add(hch)
        FC = FC.at[pi].add(fch)
        return Hs, Hsum, FC

    z = jnp.zeros((Np, H), f32)
    Hs, _, _ = jax.lax.fori_loop(0, maxd + 1, body, (z, z, z))
    return Hs[:N]
